# baseline (speedup 1.0000x reference)
import functools
import math

import jax
import jax.numpy as jnp
from jax import lax
from jax.experimental import pallas as pl
from jax.experimental.pallas import tpu as pltpu
from jax.experimental.pallas import tpu_sc as plsc

D_MODEL = 64
LANES = 16
NUM_CORES = 2
NUM_SUBCORES = 16
NUM_WORKERS = NUM_CORES * NUM_SUBCORES
CHUNK = 128
NBUF = 5


def _emb_body(toks_per_w, scale, x_hbm, table_hbm, out_hbm, idx_v, raw_v,
              scl_v, gsem, osem):
  cid = lax.axis_index("c")
  sid = lax.axis_index("s")
  wid = sid * NUM_CORES + cid
  tok0 = wid * toks_per_w

  pltpu.sync_copy(x_hbm.at[pl.ds(tok0, toks_per_w)], idx_v)

  def gather_start(c, b):
    pltpu.async_copy(table_hbm.at[idx_v.at[pl.ds(c * CHUNK, CHUNK)]],
                     raw_v.at[b], gsem.at[b])

  def gather_wait(b):
    pltpu.make_async_copy(table_hbm.at[idx_v.at[pl.ds(0, CHUNK)]],
                          raw_v.at[b], gsem.at[b]).wait()

  def out_start(c, b):
    pltpu.async_copy(scl_v.at[b],
                     out_hbm.at[pl.ds(tok0 + c * CHUNK, CHUNK),
                                pl.ds(0, D_MODEL)], osem.at[b])

  def out_wait(b):
    pltpu.make_async_copy(scl_v.at[b],
                          out_hbm.at[pl.ds(0, CHUNK), pl.ds(0, D_MODEL)],
                          osem.at[b]).wait()

  ev_idx = lax.iota(jnp.int32, LANES) * 2
  od_idx = ev_idx + 1

  for b in range(NBUF):
    gather_start(jnp.int32(b), b)

  def group(g, carry):
    for b in range(NBUF):
      c = g * NBUF + b
      gather_wait(b)

      @pl.when(g > 0)
      def _():
        out_wait(b)

      @plsc.parallel_loop(0, CHUNK, unroll=8)
      def _(r):
        rr = jnp.full((LANES,), 0, jnp.int32) + r
        for j in range(D_MODEL // (2 * LANES)):
          w = raw_v[b, r, pl.ds(j * LANES, LANES)]
          ev = lax.bitcast_convert_type(lax.shift_left(w, 16), jnp.float32)
          od = lax.bitcast_convert_type(w & jnp.int32(-65536), jnp.float32)
          plsc.store_scatter(scl_v.at[b], [rr, ev_idx + (2 * LANES) * j],
                             ev * scale)
          plsc.store_scatter(scl_v.at[b], [rr, od_idx + (2 * LANES) * j],
                             od * scale)

      out_start(c, b)

      @pl.when(c + NBUF < toks_per_w // CHUNK)
      def _():
        gather_start(c + NBUF, b)

    return carry

  lax.fori_loop(0, toks_per_w // (CHUNK * NBUF), group, 0)

  for b in range(NBUF):
    out_wait(b)


def kernel(x, table):
  bsz, seq = x.shape
  vocab, d = table.shape
  assert d == D_MODEL
  n_tok = bsz * seq
  assert n_tok % (NUM_WORKERS * CHUNK * NBUF) == 0
  toks_per_w = n_tok // NUM_WORKERS

  scale = jnp.float32(math.sqrt(d))

  table_i32 = lax.bitcast_convert_type(
      table.astype(jnp.bfloat16).reshape(vocab, d // 2, 2), jnp.int32)

  mesh = plsc.VectorSubcoreMesh(
      core_axis_name="c", subcore_axis_name="s",
      num_cores=NUM_CORES, num_subcores=NUM_SUBCORES)

  o2 = pl.kernel(
      functools.partial(_emb_body, toks_per_w, scale),
      out_type=jax.ShapeDtypeStruct((n_tok, 2 * d), jnp.float32),
      mesh=mesh,
      compiler_params=pltpu.CompilerParams(
          use_tc_tiling_on_sc=False, needs_layout_passes=False),
      scratch_types=[
          pltpu.VMEM((toks_per_w,), jnp.int32),
          pltpu.VMEM((NBUF, CHUNK, d // 2), jnp.int32),
          pltpu.VMEM((NBUF, CHUNK, d), jnp.float32),
          pltpu.SemaphoreType.DMA((NBUF,)),
          pltpu.SemaphoreType.DMA((NBUF,)),
      ],
  )(x.reshape(-1).astype(jnp.int32), table_i32)

  return o2[:, :d].reshape(bsz, seq, d)


# speedup vs baseline: 1.7840x; 1.7840x over previous
import functools
import math

import jax
import jax.numpy as jnp
from jax import lax
from jax.experimental import pallas as pl
from jax.experimental.pallas import tpu as pltpu
from jax.experimental.pallas import tpu_sc as plsc

D_MODEL = 64
LANES = 16
NUM_CORES = 2
NUM_SUBCORES = 16
NUM_WORKERS = NUM_CORES * NUM_SUBCORES
CHUNK = 128
NBUF = 5


def _emb_body(toks_per_w, scale, x_hbm, table_hbm, out_hbm, idx_v, raw_v,
              scl_v, gsem, osem):
  cid = lax.axis_index("c")
  sid = lax.axis_index("s")
  wid = sid * NUM_CORES + cid
  tok0 = wid * toks_per_w

  pltpu.sync_copy(x_hbm.at[pl.ds(tok0, toks_per_w)], idx_v)

  def gather_start(c, b):
    pltpu.async_copy(table_hbm.at[idx_v.at[pl.ds(c * CHUNK, CHUNK)]],
                     raw_v.at[b], gsem.at[b])

  def gather_wait(b):
    pltpu.make_async_copy(table_hbm.at[idx_v.at[pl.ds(0, CHUNK)]],
                          raw_v.at[b], gsem.at[b]).wait()

  def out_start(c, b):
    pltpu.async_copy(scl_v.at[b],
                     out_hbm.at[pl.ds(tok0 + c * CHUNK, CHUNK),
                                pl.ds(0, D_MODEL)], osem.at[b])

  def out_wait(b):
    pltpu.make_async_copy(scl_v.at[b],
                          out_hbm.at[pl.ds(0, CHUNK), pl.ds(0, D_MODEL)],
                          osem.at[b]).wait()

  ev_idx = lax.iota(jnp.int32, LANES) * 2
  od_idx = ev_idx + 1

  for b in range(NBUF):
    gather_start(jnp.int32(b), b)

  def group(g, carry):
    for b in range(NBUF):
      c = g * NBUF + b
      gather_wait(b)

      @pl.when(g > 0)
      def _():
        out_wait(b)

      @plsc.parallel_loop(0, CHUNK, unroll=8)
      def _(r):
        rr = jnp.full((LANES,), 0, jnp.int32) + r
        for j in range(D_MODEL // (2 * LANES)):
          w = raw_v[b, r, pl.ds(j * 2 * LANES, 2 * LANES)]
          ev, od = plsc.unpack(w, format=plsc.PackFormat.INTERLEAVED)
          plsc.store_scatter(scl_v.at[b], [rr, ev_idx + (2 * LANES) * j],
                             ev * scale)
          plsc.store_scatter(scl_v.at[b], [rr, od_idx + (2 * LANES) * j],
                             od * scale)

      out_start(c, b)

      @pl.when(c + NBUF < toks_per_w // CHUNK)
      def _():
        gather_start(c + NBUF, b)

    return carry

  lax.fori_loop(0, toks_per_w // (CHUNK * NBUF), group, 0)

  for b in range(NBUF):
    out_wait(b)


def kernel(x, table):
  bsz, seq = x.shape
  vocab, d = table.shape
  assert d == D_MODEL
  n_tok = bsz * seq
  assert n_tok % (NUM_WORKERS * CHUNK * NBUF) == 0
  toks_per_w = n_tok // NUM_WORKERS

  scale = jnp.float32(math.sqrt(d))

  table_bf = table.astype(jnp.bfloat16)

  mesh = plsc.VectorSubcoreMesh(
      core_axis_name="c", subcore_axis_name="s",
      num_cores=NUM_CORES, num_subcores=NUM_SUBCORES)

  o2 = pl.kernel(
      functools.partial(_emb_body, toks_per_w, scale),
      out_type=jax.ShapeDtypeStruct((n_tok, 2 * d), jnp.float32),
      mesh=mesh,
      compiler_params=pltpu.CompilerParams(
          use_tc_tiling_on_sc=False, needs_layout_passes=False),
      scratch_types=[
          pltpu.VMEM((toks_per_w,), jnp.int32),
          pltpu.VMEM((NBUF, CHUNK, d), jnp.bfloat16),
          pltpu.VMEM((NBUF, CHUNK, d), jnp.float32),
          pltpu.SemaphoreType.DMA((NBUF,)),
          pltpu.SemaphoreType.DMA((NBUF,)),
      ],
  )(x.reshape(-1).astype(jnp.int32), table_bf)

  return o2[:, :d].reshape(bsz, seq, d)


# speedup vs baseline: 2.1420x; 1.2007x over previous
import functools
import math

import jax
import jax.numpy as jnp
from jax import lax
from jax.experimental import pallas as pl
from jax.experimental.pallas import tpu as pltpu
from jax.experimental.pallas import tpu_sc as plsc

D_MODEL = 64
LANES = 16
NUM_CORES = 2
NUM_SUBCORES = 16
NUM_WORKERS = NUM_CORES * NUM_SUBCORES
CHUNK = 128
NBUF = 5


def _emb_body(toks_per_w, scale, x_hbm, table_hbm, out_hbm, idx_v, raw_v,
              scl_v, gsem, osem):
  cid = lax.axis_index("c")
  sid = lax.axis_index("s")
  wid = sid * NUM_CORES + cid
  tok0 = wid * toks_per_w

  pltpu.sync_copy(x_hbm.at[pl.ds(tok0, toks_per_w)], idx_v)

  def gather_start(c, b):
    pltpu.async_copy(table_hbm.at[idx_v.at[pl.ds(c * CHUNK, CHUNK)]],
                     raw_v.at[b], gsem.at[b])

  def gather_wait(b):
    pltpu.make_async_copy(table_hbm.at[idx_v.at[pl.ds(0, CHUNK)]],
                          raw_v.at[b], gsem.at[b]).wait()

  def out_start(c, b):
    pltpu.async_copy(scl_v.at[b],
                     out_hbm.at[pl.ds(tok0 + c * CHUNK, CHUNK),
                                pl.ds(0, D_MODEL)], osem.at[b])

  def out_wait(b):
    pltpu.make_async_copy(scl_v.at[b],
                          out_hbm.at[pl.ds(0, CHUNK), pl.ds(0, D_MODEL)],
                          osem.at[b]).wait()

  for b in range(NBUF):
    gather_start(jnp.int32(b), b)

  def group(g, carry):
    for b in range(NBUF):
      c = g * NBUF + b
      gather_wait(b)

      @pl.when(g > 0)
      def _():
        out_wait(b)

      @plsc.parallel_loop(0, CHUNK, unroll=8)
      def _(r):
        for j in range(D_MODEL // LANES):
          sl = pl.ds(j * LANES, LANES)
          scl_v[b, r, sl] = raw_v[b, r, sl] * scale

      out_start(c, b)

      @pl.when(c + NBUF < toks_per_w // CHUNK)
      def _():
        gather_start(c + NBUF, b)

    return carry

  lax.fori_loop(0, toks_per_w // (CHUNK * NBUF), group, 0)

  for b in range(NBUF):
    out_wait(b)


def kernel(x, table):
  bsz, seq = x.shape
  vocab, d = table.shape
  assert d == D_MODEL
  n_tok = bsz * seq
  assert n_tok % (NUM_WORKERS * CHUNK * NBUF) == 0
  toks_per_w = n_tok // NUM_WORKERS

  scale = jnp.float32(math.sqrt(d))

  mesh = plsc.VectorSubcoreMesh(
      core_axis_name="c", subcore_axis_name="s",
      num_cores=NUM_CORES, num_subcores=NUM_SUBCORES)

  o2 = pl.kernel(
      functools.partial(_emb_body, toks_per_w, scale),
      out_type=jax.ShapeDtypeStruct((n_tok, 2 * d), jnp.float32),
      mesh=mesh,
      compiler_params=pltpu.CompilerParams(use_tc_tiling_on_sc=False),
      scratch_types=[
          pltpu.VMEM((toks_per_w,), jnp.int32),
          pltpu.VMEM((NBUF, CHUNK, d), jnp.float32),
          pltpu.VMEM((NBUF, CHUNK, d), jnp.float32),
          pltpu.SemaphoreType.DMA((NBUF,)),
          pltpu.SemaphoreType.DMA((NBUF,)),
      ],
  )(x.reshape(-1).astype(jnp.int32), table)

  return o2[:, :d].reshape(bsz, seq, d)
